# Initial kernel scaffold; baseline (speedup 1.0000x reference)
import jax
import jax.numpy as jnp
from jax.experimental import pallas as pl


def kernel(z, theta):
    raise NotImplementedError("write your pallas kernel here")


# speedup vs baseline: 1977.9185x; 1977.9185x over previous
import dataclasses
import functools
import math

import jax
import jax.numpy as jnp
from jax import lax
from jax.experimental import pallas as pl
from jax.experimental.pallas import tpu as pltpu
from jax.experimental.pallas import tpu_sc as plsc

_DEPTH = 12
_LEAVES = 1 << _DEPTH
_NODES2 = 2 * ((1 << _DEPTH) - 1)
_NODES2_PAD = 8192
_B = 1_000_000
_NC, _NS, _L = 2, 16, 16
_NW = _NC * _NS
_CHUNK = 31_248
_REM_BASE = _NW * _CHUNK
_REM = _B - _REM_BASE
_LOG2X12 = jnp.float32(_DEPTH * math.log(2.0))


def _log_body(x_ref, o_ref):
    o_ref[...] = jnp.log(x_ref[...] + 1e-20)


def _log_theta(theta):
    tf = jnp.reshape(theta, (-1,))
    tf = jnp.pad(tf, (0, _NODES2_PAD - _NODES2), constant_values=1.0)
    lt = pl.pallas_call(
        _log_body,
        out_shape=jax.ShapeDtypeStruct((_NODES2_PAD // 128, 128), jnp.float32),
    )(tf.reshape(_NODES2_PAD // 128, 128))
    return lt.reshape(_NODES2_PAD)


def _gather_16(tab_v, z16):
    zc = jnp.minimum(jnp.maximum(z16, jnp.float32(0.0)), jnp.float32(1.0 - 1e-8))
    leaf = (zc * jnp.float32(_LEAVES)).astype(jnp.int32)
    leaf = jnp.minimum(leaf, _LEAVES - 1)
    return plsc.load_gather(tab_v, [leaf])


def _sc_body(lt_hbm, z_hbm, out_hbm, lt_v, tab_v, z_v, o_v, sem):
    wid = lax.axis_index("s") * _NC + lax.axis_index("c")
    base = wid * _CHUNK
    zcopy = pltpu.async_copy(z_hbm.at[pl.ds(base, _CHUNK)], z_v.at[pl.ds(0, _CHUNK)], sem)
    pltpu.sync_copy(lt_hbm, lt_v)

    lanes = lax.iota(jnp.int32, _L)

    def tab_body(i, carry):
        j = lanes + i * _L
        acc = jnp.full((_L,), _LOG2X12, jnp.float32)
        for l in range(_DEPTH):
            idx = (j >> (11 - l)) + ((1 << (l + 1)) - 2)
            acc = acc + plsc.load_gather(lt_v, [idx])
        tab_v[pl.ds(i * _L, _L)] = acc
        return carry

    lax.fori_loop(0, _LEAVES // _L, tab_body, 0)
    zcopy.wait()

    is_last = wid == _NW - 1

    @pl.when(is_last)
    def _():
        pltpu.sync_copy(z_hbm.at[pl.ds(_REM_BASE, _REM)], z_v.at[pl.ds(_CHUNK, _REM)])

    def body(i, carry):
        o_v[pl.ds(i * _L, _L)] = _gather_16(tab_v, z_v[pl.ds(i * _L, _L)])
        return carry

    n_iters = jnp.where(is_last, (_CHUNK + _REM) // _L, _CHUNK // _L)
    lax.fori_loop(0, n_iters, body, 0)

    pltpu.sync_copy(o_v.at[pl.ds(0, _CHUNK)], out_hbm.at[pl.ds(base, _CHUNK)])

    @pl.when(is_last)
    def _():
        pltpu.sync_copy(o_v.at[pl.ds(_CHUNK, _REM)], out_hbm.at[pl.ds(_REM_BASE, _REM)])


_CP = pltpu.CompilerParams()
if "needs_layout_passes" in pltpu.CompilerParams.__dataclass_fields__:
    _CP = dataclasses.replace(_CP, needs_layout_passes=False)


@functools.partial(
    pl.kernel,
    mesh=plsc.VectorSubcoreMesh(core_axis_name="c", subcore_axis_name="s"),
    compiler_params=_CP,
    out_type=jax.ShapeDtypeStruct((_B,), jnp.float32),
    scratch_types=[
        pltpu.VMEM((_NODES2_PAD,), jnp.float32),
        pltpu.VMEM((_LEAVES,), jnp.float32),
        pltpu.VMEM((_CHUNK + _REM,), jnp.float32),
        pltpu.VMEM((_CHUNK + _REM,), jnp.float32),
        pltpu.SemaphoreType.DMA,
    ],
)
def _sc_kernel(lt_hbm, z_hbm, out_hbm, lt_v, tab_v, z_v, o_v, sem):
    _sc_body(lt_hbm, z_hbm, out_hbm, lt_v, tab_v, z_v, o_v, sem)


def kernel(z, theta):
    return _sc_kernel(_log_theta(theta), z)


# speedup vs baseline: 3866.5022x; 1.9548x over previous
import dataclasses
import functools
import math

import jax
import jax.numpy as jnp
from jax import lax
from jax.experimental import pallas as pl
from jax.experimental.pallas import tpu as pltpu
from jax.experimental.pallas import tpu_sc as plsc

_DEPTH = 12
_LEAVES = 1 << _DEPTH
_NODES2 = 2 * ((1 << _DEPTH) - 1)
_NODES2_PAD = 8192
_B = 1_000_000
_NC, _NS, _L = 2, 16, 16
_NW = _NC * _NS
_CHUNK = 31_264
_LOG2X12 = jnp.float32(_DEPTH * math.log(2.0))


def _log_body(x_ref, o_ref):
    o_ref[...] = jnp.log(x_ref[...] + 1e-20)


def _log_theta(theta):
    tf = jnp.reshape(theta, (-1,))
    tf = jnp.pad(tf, (0, _NODES2_PAD - _NODES2), constant_values=1.0)
    lt = pl.pallas_call(
        _log_body,
        out_shape=jax.ShapeDtypeStruct((_NODES2_PAD // 128, 128), jnp.float32),
    )(tf.reshape(_NODES2_PAD // 128, 128))
    return lt.reshape(_NODES2_PAD)


def _gather_16(tab_v, z16):
    zc = jnp.minimum(jnp.maximum(z16, jnp.float32(0.0)), jnp.float32(1.0 - 1e-8))
    leaf = (zc * jnp.float32(_LEAVES)).astype(jnp.int32)
    leaf = jnp.minimum(leaf, _LEAVES - 1)
    return plsc.load_gather(tab_v, [leaf])


def _sc_body(lt_hbm, z_hbm, out_hbm, lt_v, tab_v, z_v, o_v, sem):
    wid = lax.axis_index("s") * _NC + lax.axis_index("c")
    base = jnp.minimum(wid * _CHUNK, _B - _CHUNK)
    zcopy = pltpu.async_copy(z_hbm.at[pl.ds(base, _CHUNK)], z_v, sem)
    pltpu.sync_copy(lt_hbm, lt_v)

    lanes = lax.iota(jnp.int32, _L)

    @plsc.parallel_loop(0, _LEAVES, step=_L, unroll=4)
    def _(i):
        j = lanes + i
        acc = jnp.full((_L,), _LOG2X12, jnp.float32)
        for l in range(_DEPTH):
            idx = (j >> (11 - l)) + ((1 << (l + 1)) - 2)
            acc = acc + plsc.load_gather(lt_v, [idx])
        tab_v[pl.ds(i, _L)] = acc

    zcopy.wait()

    @plsc.parallel_loop(0, _CHUNK, step=_L, unroll=8)
    def _(i):
        o_v[pl.ds(i, _L)] = _gather_16(tab_v, z_v[pl.ds(i, _L)])

    pltpu.sync_copy(o_v, out_hbm.at[pl.ds(base, _CHUNK)])


_CP = pltpu.CompilerParams()
if "needs_layout_passes" in pltpu.CompilerParams.__dataclass_fields__:
    _CP = dataclasses.replace(_CP, needs_layout_passes=False)


@functools.partial(
    pl.kernel,
    mesh=plsc.VectorSubcoreMesh(core_axis_name="c", subcore_axis_name="s"),
    compiler_params=_CP,
    out_type=jax.ShapeDtypeStruct((_B,), jnp.float32),
    scratch_types=[
        pltpu.VMEM((_NODES2_PAD,), jnp.float32),
        pltpu.VMEM((_LEAVES,), jnp.float32),
        pltpu.VMEM((_CHUNK,), jnp.float32),
        pltpu.VMEM((_CHUNK,), jnp.float32),
        pltpu.SemaphoreType.DMA,
    ],
)
def _sc_kernel(lt_hbm, z_hbm, out_hbm, lt_v, tab_v, z_v, o_v, sem):
    _sc_body(lt_hbm, z_hbm, out_hbm, lt_v, tab_v, z_v, o_v, sem)


def kernel(z, theta):
    return _sc_kernel(_log_theta(theta), z)
